# baseline (speedup 1.0000x reference)
import functools

import jax
import jax.numpy as jnp
from jax import lax
from jax.experimental import pallas as pl
from jax.experimental.pallas import tpu as pltpu
from jax.experimental.pallas import tpu_sc as plsc

N_NODE = 10000
D_IN = 256
E_TOT = 160000
D_H = 512
D_HW = D_H // 2
D_OUT = 4
LANES = 128


_BLK_N = 1000


def _pack_pairs(x_f32):
    bits = lax.bitcast_convert_type(x_f32.astype(jnp.bfloat16), jnp.uint16)
    bits = bits.astype(jnp.uint32)
    word = bits[:, :D_HW] | (bits[:, D_HW:] << 16)
    return lax.bitcast_convert_type(word, jnp.int32)


def _pre_body(xu_ref, xi_ref, w1a_ref, w1b_ref, b1_ref, u_ref, v_ref):
    u = jnp.dot(xu_ref[...], w1a_ref[...], preferred_element_type=jnp.float32)
    u_ref[...] = _pack_pairs(u + b1_ref[...])
    v = jnp.dot(xi_ref[...], w1b_ref[...], preferred_element_type=jnp.float32)
    v_ref[...] = _pack_pairs(v)


def _precompute(x_user, x_item, w1a, w1b, b1_2d):
    grid = (N_NODE // _BLK_N,)
    return pl.pallas_call(
        _pre_body,
        grid=grid,
        in_specs=[
            pl.BlockSpec((_BLK_N, D_IN), lambda i: (i, 0)),
            pl.BlockSpec((_BLK_N, D_IN), lambda i: (i, 0)),
            pl.BlockSpec((D_IN, D_H), lambda i: (0, 0)),
            pl.BlockSpec((D_IN, D_H), lambda i: (0, 0)),
            pl.BlockSpec((1, D_H), lambda i: (0, 0)),
        ],
        out_specs=[
            pl.BlockSpec((_BLK_N, D_HW), lambda i: (i, 0)),
            pl.BlockSpec((_BLK_N, D_HW), lambda i: (i, 0)),
        ],
        out_shape=[
            jax.ShapeDtypeStruct((N_NODE, D_HW), jnp.int32),
            jax.ShapeDtypeStruct((N_NODE, D_HW), jnp.int32),
        ],
    )(x_user, x_item, w1a, w1b, b1_2d)


_INFO = plsc.get_sparse_core_info()
_NC = _INFO.num_cores
_NS = _INFO.num_subcores
_NW = _NC * _NS
_CH = 64

_sc_mesh = plsc.VectorSubcoreMesh(core_axis_name="c", subcore_axis_name="s")


def _make_gather_add(e_tot):
    per_w = e_tot // _NW
    nch = per_w // _CH

    @functools.partial(
        pl.kernel,
        out_type=jax.ShapeDtypeStruct((e_tot, D_HW), jnp.int32),
        mesh=_sc_mesh,
        scratch_types=[
            pltpu.VMEM((per_w,), jnp.int32),
            pltpu.VMEM((per_w,), jnp.int32),
            pltpu.VMEM((2 * _CH, D_HW), jnp.int32),
            pltpu.VMEM((2 * _CH, D_HW), jnp.int32),
            pltpu.SemaphoreType.DMA,
            pltpu.SemaphoreType.DMA,
            pltpu.SemaphoreType.DMA,
            pltpu.SemaphoreType.DMA,
        ],
    )
    def _gather_add(u_hbm, v_hbm, row_hbm, col_hbm, out_hbm,
                    idxr, idxc, buf_a, buf_b, sg_a, sg_b, ss_a, ss_b):
        wid = lax.axis_index("s") * _NC + lax.axis_index("c")
        base = wid * per_w
        pltpu.sync_copy(row_hbm.at[pl.ds(base, per_w)], idxr)
        pltpu.sync_copy(col_hbm.at[pl.ds(base, per_w)], idxc)

        def g_copies(i, buf, sem):
            off = i * _CH
            return (
                pltpu.make_async_copy(
                    u_hbm.at[idxr.at[pl.ds(off, _CH)]], buf.at[pl.ds(0, _CH)], sem),
                pltpu.make_async_copy(
                    v_hbm.at[idxc.at[pl.ds(off, _CH)]], buf.at[pl.ds(_CH, _CH)], sem),
            )

        def start_g(i, buf, sem):
            cu, cv = g_copies(i, buf, sem)
            cu.start()
            cv.start()

        def wait_g(i, buf, sem):
            cu, cv = g_copies(i, buf, sem)
            cu.wait()
            cv.wait()

        def s_copy(i, buf, sem):
            off = i * _CH
            return pltpu.make_async_copy(
                buf.at[pl.ds(0, _CH)], out_hbm.at[pl.ds(base + off, _CH)], sem)

        def add_rows(buf):
            def row_body(j, c2):
                for k in range(D_HW // 16):
                    sl = pl.ds(k * 16, 16)
                    wu = buf[j, sl]
                    wv = buf[j + _CH, sl]
                    lo = lax.bitcast_convert_type(wu << 16, jnp.float32) + \
                        lax.bitcast_convert_type(wv << 16, jnp.float32)
                    hi = lax.bitcast_convert_type(wu, jnp.float32) + \
                        lax.bitcast_convert_type(wv, jnp.float32)
                    lo_b = lax.shift_right_logical(
                        lax.bitcast_convert_type(lo, jnp.int32), 16)
                    hi_b = lax.bitcast_convert_type(hi, jnp.int32) & -65536
                    buf[j, sl] = lo_b | hi_b
                return c2

            lax.fori_loop(0, _CH, row_body, 0)

        start_g(0, buf_a, sg_a)

        def body(g, carry):
            i0 = 2 * g
            i1 = i0 + 1

            @pl.when(g > 0)
            def _():
                s_copy(i1 - 2, buf_b, ss_b).wait()

            start_g(i1, buf_b, sg_b)
            wait_g(i0, buf_a, sg_a)
            add_rows(buf_a)
            s_copy(i0, buf_a, ss_a).start()
            wait_g(i1, buf_b, sg_b)
            add_rows(buf_b)
            s_copy(i0, buf_a, ss_a).wait()

            @pl.when(i0 + 2 < nch)
            def _():
                start_g(i0 + 2, buf_a, sg_a)

            s_copy(i1, buf_b, ss_b).start()
            return carry

        lax.fori_loop(0, nch // 2, body, 0)
        s_copy(nch - 1, buf_b, ss_b).wait()

    return _gather_add


_N_SPLIT = 5
_E_CHUNK = 32768
_E_PAD = _N_SPLIT * _E_CHUNK
_gather_add_chunk = _make_gather_add(_E_CHUNK)


_BLK_E = 2048


def _mlp_body(h1_ref, w2a_ref, w2b_ref, b2_ref, w3_ref, b3_ref, out_ref):
    w = h1_ref[...]
    h_lo = lax.bitcast_convert_type(w << 16, jnp.float32)
    h_hi = lax.bitcast_convert_type(
        lax.bitcast_convert_type(w, jnp.uint32) & jnp.uint32(0xFFFF0000),
        jnp.float32,
    )
    h_lo = jnp.maximum(h_lo, 0.0).astype(jnp.bfloat16)
    h_hi = jnp.maximum(h_hi, 0.0).astype(jnp.bfloat16)
    a2 = (
        jnp.dot(h_lo, w2a_ref[...], preferred_element_type=jnp.float32)
        + jnp.dot(h_hi, w2b_ref[...], preferred_element_type=jnp.float32)
        + b2_ref[...]
    )
    a2 = jnp.maximum(a2, 0.0).astype(jnp.bfloat16)
    lg = jnp.dot(a2, w3_ref[...], preferred_element_type=jnp.float32) + b3_ref[...]
    m = jnp.max(lg, axis=1, keepdims=True)
    s = jnp.sum(jnp.exp(lg - m), axis=1, keepdims=True)
    out_ref[...] = (lg - m - jnp.log(s))[:, :D_OUT]


def _mlp(h1, w2a, w2b, b2_2d, w3p, b3p):
    n_e = h1.shape[0]
    grid = (n_e // _BLK_E,)
    return pl.pallas_call(
        _mlp_body,
        grid=grid,
        in_specs=[
            pl.BlockSpec((_BLK_E, D_HW), lambda i: (i, 0)),
            pl.BlockSpec((D_HW, D_H), lambda i: (0, 0)),
            pl.BlockSpec((D_HW, D_H), lambda i: (0, 0)),
            pl.BlockSpec((1, D_H), lambda i: (0, 0)),
            pl.BlockSpec((D_H, LANES), lambda i: (0, 0)),
            pl.BlockSpec((1, LANES), lambda i: (0, 0)),
        ],
        out_specs=pl.BlockSpec((_BLK_E, D_OUT), lambda i: (i, 0)),
        out_shape=jax.ShapeDtypeStruct((n_e, D_OUT), jnp.float32),
    )(h1, w2a, w2b, b2_2d, w3p, b3p)


def kernel(x_user, x_item, edge_index, W1, b1, W2, b2, W3, b3):
    w1a = W1[:D_IN]
    w1b = W1[D_IN:]
    row = edge_index[0].astype(jnp.int32)
    col = edge_index[1].astype(jnp.int32)
    pad = jnp.zeros((_E_PAD - E_TOT,), jnp.int32)
    row = jnp.concatenate([row, pad])
    col = jnp.concatenate([col, pad])

    u, v = _precompute(x_user, x_item, w1a, w1b, b1.reshape(1, D_H))

    w2a = W2[:D_HW].astype(jnp.bfloat16)
    w2b = W2[D_HW:].astype(jnp.bfloat16)
    w3p = jnp.zeros((D_H, LANES), jnp.float32).at[:, :D_OUT].set(W3)
    w3p = w3p.astype(jnp.bfloat16)
    b3p = jnp.full((1, LANES), -1e30, jnp.float32).at[0, :D_OUT].set(b3)
    b2_2d = b2.reshape(1, D_H)

    outs = []
    for c in range(_N_SPLIT):
        lo = c * _E_CHUNK
        h1 = _gather_add_chunk(u, v, row[lo:lo + _E_CHUNK], col[lo:lo + _E_CHUNK])
        outs.append(_mlp(h1, w2a, w2b, b2_2d, w3p, b3p))
    return jnp.concatenate(outs, axis=0)[:E_TOT]


# speedup vs baseline: 1.3986x; 1.3986x over previous
import functools

import jax
import jax.numpy as jnp
from jax import lax
from jax.experimental import pallas as pl
from jax.experimental.pallas import tpu as pltpu
from jax.experimental.pallas import tpu_sc as plsc

N_NODE = 10000
D_IN = 256
E_TOT = 160000
D_H = 512
D_HW = D_H // 2
D_OUT = 4
LANES = 128


_BLK_N = 1000


def _pack_pairs(x_f32):
    bits = lax.bitcast_convert_type(x_f32.astype(jnp.bfloat16), jnp.uint16)
    bits = bits.astype(jnp.uint32)
    word = bits[:, :D_HW] | (bits[:, D_HW:] << 16)
    return lax.bitcast_convert_type(word, jnp.int32)


def _pre_body(xu_ref, xi_ref, w1a_ref, w1b_ref, b1_ref, u_ref, v_ref):
    u = jnp.dot(xu_ref[...], w1a_ref[...], preferred_element_type=jnp.float32)
    u_ref[...] = _pack_pairs(u + b1_ref[...])
    v = jnp.dot(xi_ref[...], w1b_ref[...], preferred_element_type=jnp.float32)
    v_ref[...] = _pack_pairs(v)


def _precompute(x_user, x_item, w1a, w1b, b1_2d):
    grid = (N_NODE // _BLK_N,)
    return pl.pallas_call(
        _pre_body,
        grid=grid,
        in_specs=[
            pl.BlockSpec((_BLK_N, D_IN), lambda i: (i, 0)),
            pl.BlockSpec((_BLK_N, D_IN), lambda i: (i, 0)),
            pl.BlockSpec((D_IN, D_H), lambda i: (0, 0)),
            pl.BlockSpec((D_IN, D_H), lambda i: (0, 0)),
            pl.BlockSpec((1, D_H), lambda i: (0, 0)),
        ],
        out_specs=[
            pl.BlockSpec((_BLK_N, D_HW), lambda i: (i, 0)),
            pl.BlockSpec((_BLK_N, D_HW), lambda i: (i, 0)),
        ],
        out_shape=[
            jax.ShapeDtypeStruct((N_NODE, D_HW), jnp.int32),
            jax.ShapeDtypeStruct((N_NODE, D_HW), jnp.int32),
        ],
    )(x_user, x_item, w1a, w1b, b1_2d)


_INFO = plsc.get_sparse_core_info()
_NC = _INFO.num_cores
_NS = _INFO.num_subcores
_NW = _NC * _NS
_CH = 40

_sc_mesh = plsc.VectorSubcoreMesh(core_axis_name="c", subcore_axis_name="s")


def _make_gather_add(e_tot):
    per_w = e_tot // _NW
    nch = per_w // _CH

    @functools.partial(
        pl.kernel,
        out_type=jax.ShapeDtypeStruct((e_tot, D_HW), jnp.int32),
        mesh=_sc_mesh,
        scratch_types=[
            pltpu.VMEM((per_w,), jnp.int32),
            pltpu.VMEM((per_w,), jnp.int32),
            pltpu.VMEM((2 * _CH, D_HW), jnp.int32),
            pltpu.VMEM((2 * _CH, D_HW), jnp.int32),
            pltpu.SemaphoreType.DMA,
            pltpu.SemaphoreType.DMA,
            pltpu.SemaphoreType.DMA,
            pltpu.SemaphoreType.DMA,
        ],
    )
    def _gather_add(u_hbm, v_hbm, row_hbm, col_hbm, out_hbm,
                    idxr, idxc, buf_a, buf_b, sg_a, sg_b, ss_a, ss_b):
        wid = lax.axis_index("s") * _NC + lax.axis_index("c")
        base = wid * per_w
        pltpu.sync_copy(row_hbm.at[pl.ds(base, per_w)], idxr)
        pltpu.sync_copy(col_hbm.at[pl.ds(base, per_w)], idxc)

        def g_copies(i, buf, sem):
            off = i * _CH
            return (
                pltpu.make_async_copy(
                    u_hbm.at[idxr.at[pl.ds(off, _CH)]], buf.at[pl.ds(0, _CH)], sem),
                pltpu.make_async_copy(
                    v_hbm.at[idxc.at[pl.ds(off, _CH)]], buf.at[pl.ds(_CH, _CH)], sem),
            )

        def start_g(i, buf, sem):
            cu, cv = g_copies(i, buf, sem)
            cu.start()
            cv.start()

        def wait_g(i, buf, sem):
            cu, cv = g_copies(i, buf, sem)
            cu.wait()
            cv.wait()

        def s_copy(i, buf, sem):
            off = i * _CH
            return pltpu.make_async_copy(
                buf.at[pl.ds(0, _CH)], out_hbm.at[pl.ds(base + off, _CH)], sem)

        def add_rows(buf):
            def row_body(j, c2):
                for k in range(D_HW // 16):
                    sl = pl.ds(k * 16, 16)
                    wu = buf[j, sl]
                    wv = buf[j + _CH, sl]
                    lo = lax.bitcast_convert_type(wu << 16, jnp.float32) + \
                        lax.bitcast_convert_type(wv << 16, jnp.float32)
                    hi = lax.bitcast_convert_type(wu, jnp.float32) + \
                        lax.bitcast_convert_type(wv, jnp.float32)
                    lo_b = lax.shift_right_logical(
                        lax.bitcast_convert_type(lo, jnp.int32), 16)
                    hi_b = lax.bitcast_convert_type(hi, jnp.int32) & -65536
                    buf[j, sl] = lo_b | hi_b
                return c2

            lax.fori_loop(0, _CH, row_body, 0)

        start_g(0, buf_a, sg_a)

        def body(g, carry):
            i0 = 2 * g
            i1 = i0 + 1

            @pl.when(g > 0)
            def _():
                s_copy(i1 - 2, buf_b, ss_b).wait()

            start_g(i1, buf_b, sg_b)
            wait_g(i0, buf_a, sg_a)
            add_rows(buf_a)
            s_copy(i0, buf_a, ss_a).start()
            wait_g(i1, buf_b, sg_b)
            add_rows(buf_b)
            s_copy(i0, buf_a, ss_a).wait()
            start_g(i0 + 2, buf_a, sg_a)
            s_copy(i1, buf_b, ss_b).start()
            return carry

        lax.fori_loop(0, (nch - 1) // 2, body, 0)

        last = nch - 1
        wait_g(last, buf_a, sg_a)
        add_rows(buf_a)
        s_copy(last, buf_a, ss_a).start()
        s_copy(last - 1, buf_b, ss_b).wait()
        s_copy(last, buf_a, ss_a).wait()

    return _gather_add


_N_SPLIT = 5
_E_CHUNK = E_TOT // _N_SPLIT
_gather_add_chunk = _make_gather_add(_E_CHUNK)


_BLK_E = 3200


def _mlp_body(h1_ref, w2a_ref, w2b_ref, b2_ref, w3_ref, b3_ref, out_ref):
    w = h1_ref[...]
    h_lo = lax.bitcast_convert_type(w << 16, jnp.float32)
    h_hi = lax.bitcast_convert_type(
        lax.bitcast_convert_type(w, jnp.uint32) & jnp.uint32(0xFFFF0000),
        jnp.float32,
    )
    h_lo = jnp.maximum(h_lo, 0.0).astype(jnp.bfloat16)
    h_hi = jnp.maximum(h_hi, 0.0).astype(jnp.bfloat16)
    a2 = (
        jnp.dot(h_lo, w2a_ref[...], preferred_element_type=jnp.float32)
        + jnp.dot(h_hi, w2b_ref[...], preferred_element_type=jnp.float32)
        + b2_ref[...]
    )
    a2 = jnp.maximum(a2, 0.0).astype(jnp.bfloat16)
    lg = jnp.dot(a2, w3_ref[...], preferred_element_type=jnp.float32) + b3_ref[...]
    m = jnp.max(lg, axis=1, keepdims=True)
    s = jnp.sum(jnp.exp(lg - m), axis=1, keepdims=True)
    out_ref[...] = (lg - m - jnp.log(s))[:, :D_OUT]


def _mlp(h1, w2a, w2b, b2_2d, w3p, b3p):
    n_e = h1.shape[0]
    grid = (n_e // _BLK_E,)
    return pl.pallas_call(
        _mlp_body,
        grid=grid,
        in_specs=[
            pl.BlockSpec((_BLK_E, D_HW), lambda i: (i, 0)),
            pl.BlockSpec((D_HW, D_H), lambda i: (0, 0)),
            pl.BlockSpec((D_HW, D_H), lambda i: (0, 0)),
            pl.BlockSpec((1, D_H), lambda i: (0, 0)),
            pl.BlockSpec((D_H, LANES), lambda i: (0, 0)),
            pl.BlockSpec((1, LANES), lambda i: (0, 0)),
        ],
        out_specs=pl.BlockSpec((_BLK_E, D_OUT), lambda i: (i, 0)),
        out_shape=jax.ShapeDtypeStruct((n_e, D_OUT), jnp.float32),
    )(h1, w2a, w2b, b2_2d, w3p, b3p)


def kernel(x_user, x_item, edge_index, W1, b1, W2, b2, W3, b3):
    w1a = W1[:D_IN]
    w1b = W1[D_IN:]
    row = edge_index[0].astype(jnp.int32)
    col = edge_index[1].astype(jnp.int32)

    u, v = _precompute(x_user, x_item, w1a, w1b, b1.reshape(1, D_H))

    w2a = W2[:D_HW].astype(jnp.bfloat16)
    w2b = W2[D_HW:].astype(jnp.bfloat16)
    w3p = jnp.zeros((D_H, LANES), jnp.float32).at[:, :D_OUT].set(W3)
    w3p = w3p.astype(jnp.bfloat16)
    b3p = jnp.full((1, LANES), -1e30, jnp.float32).at[0, :D_OUT].set(b3)
    b2_2d = b2.reshape(1, D_H)

    outs = []
    for c in range(_N_SPLIT):
        lo = c * _E_CHUNK
        h1 = _gather_add_chunk(u, v, row[lo:lo + _E_CHUNK], col[lo:lo + _E_CHUNK])
        outs.append(_mlp(h1, w2a, w2b, b2_2d, w3p, b3p))
    return jnp.concatenate(outs, axis=0)
